# baseline (speedup 1.0000x reference)
import functools

import jax
import jax.numpy as jnp
from jax import lax
from jax.experimental import pallas as pl
from jax.experimental.pallas import tpu as pltpu
from jax.experimental.pallas import tpu_sc as plsc

B = 8
H = 224
W = 224
C = 96
NC = 2
NS = 16
NW = NC * NS
ROWS_PER_W = H * B // NW
CHUNK = 32
PARTS = W // CHUNK
TW = 128
CG = C // 16
NCHUNK = PARTS * ROWS_PER_W


def _bcast_f32(x):
    return lax.broadcast_in_dim(x, (16,), ())


def _body(img_hbm, theta_hbm, out_hbm, theta_v, idx_v, w_v, rows_v, outb_v,
          gsem0, gsem1, osem0, osem1):
    wid = lax.axis_index("s") * NC + lax.axis_index("c")
    b = wid // 4
    sub = wid % 4

    pltpu.sync_copy(theta_hbm, theta_v)
    tv = theta_v[b, :]
    t0 = _bcast_f32(tv[0])
    t1 = _bcast_f32(tv[1])
    t2 = _bcast_f32(tv[2])
    t3 = _bcast_f32(tv[3])
    t4 = _bcast_f32(tv[4])
    t5 = _bcast_f32(tv[5])

    lane = lax.iota(jnp.int32, 16)
    bbase = b * (H * W)
    bbase_v = lax.broadcast_in_dim(bbase, (16,), ())
    maxc = 223.0
    gsems = (gsem0, gsem1)
    osems = (osem0, osem1)

    def produce(cc, buf):
        i = (cc // PARTS) * 4 + sub
        part = cc % PARTS
        iv = lax.broadcast_in_dim(i, (16,), ())
        ifv = iv.astype(jnp.float32)
        jbase = part * CHUNK
        for g in range(CHUNK // 16):
            jv = lane + jbase + g * 16
            jf = jv.astype(jnp.float32)
            ys = t0 * ifv + t1 * jf + t2
            xs = t3 * ifv + t4 * jf + t5
            ys0 = jnp.maximum(ys, 0.0)
            xs0 = jnp.maximum(xs, 0.0)
            yl = ys0.astype(jnp.int32)
            xl = xs0.astype(jnp.int32)
            dy = ys0 - yl.astype(jnp.float32)
            dx = xs0 - xl.astype(jnp.float32)
            ylc = jnp.minimum(yl, 223)
            yuc = jnp.minimum(yl + 1, 223)
            xlc = jnp.minimum(xl, 223)
            xuc = jnp.minimum(xl + 1, 223)
            ok = ((ys >= 0.0) & (ys <= maxc)) & ((xs >= 0.0) & (xs <= maxc))
            m = jnp.where(ok, 1.0, 0.0).astype(jnp.float32)
            ny = 1.0 - dy
            nx = 1.0 - dx
            r0 = bbase_v + ylc * W
            r1 = bbase_v + yuc * W
            s = pl.ds(g * 16, 16)
            idx_v[buf, 0, s] = r0 + xlc
            idx_v[buf, 1, s] = r0 + xuc
            idx_v[buf, 2, s] = r1 + xlc
            idx_v[buf, 3, s] = r1 + xuc
            w_v[buf, 0, s] = nx * ny * m
            w_v[buf, 1, s] = dx * ny * m
            w_v[buf, 2, s] = nx * dy * m
            w_v[buf, 3, s] = dx * dy * m
        for c in range(4):
            pltpu.async_copy(
                img_hbm.at[idx_v.at[buf, c]],
                rows_v.at[buf, pl.ds(c * CHUNK, CHUNK)],
                gsems[buf],
            )

    def consume(cc, buf):
        @pl.when(cc >= 2)
        def _():
            pltpu.make_async_copy(
                outb_v.at[buf], out_hbm.at[pl.ds(0, CHUNK)], osems[buf]
            ).wait()

        for c in range(4):
            pltpu.make_async_copy(
                img_hbm.at[idx_v.at[buf, c]],
                rows_v.at[buf, pl.ds(c * CHUNK, CHUNK)],
                gsems[buf],
            ).wait()

        @pl.loop(0, CHUNK // 16)
        def _grp(g):
            base = g * 16
            ws = pl.ds(base, 16)
            w0v = w_v[buf, 0, ws]
            w1v = w_v[buf, 1, ws]
            w2v = w_v[buf, 2, ws]
            w3v = w_v[buf, 3, ws]
            for p16 in range(16):
                p = base + p16
                w0 = w0v[p16]
                w1 = w1v[p16]
                w2 = w2v[p16]
                w3 = w3v[p16]
                for cg in range(CG):
                    cs = pl.ds(cg * 16, 16)
                    v = (rows_v[buf, p, cs] * w0
                         + rows_v[buf, CHUNK + p, cs] * w1
                         + rows_v[buf, 2 * CHUNK + p, cs] * w2
                         + rows_v[buf, 3 * CHUNK + p, cs] * w3)
                    outb_v[buf, p, cs] = v

        i = (cc // PARTS) * 4 + sub
        part = cc % PARTS
        gbase = bbase + i * W + part * CHUNK
        pltpu.async_copy(
            outb_v.at[buf], out_hbm.at[pl.ds(gbase, CHUNK)], osems[buf]
        )

    produce(0, 0)

    @pl.loop(0, NCHUNK // 2 - 1)
    def _t(t):
        cc = 2 * t
        produce(cc + 1, 1)
        consume(cc, 0)
        produce(cc + 2, 0)
        consume(cc + 1, 1)

    produce(NCHUNK - 1, 1)
    consume(NCHUNK - 2, 0)
    consume(NCHUNK - 1, 1)

    for buf in range(2):
        pltpu.make_async_copy(
            outb_v.at[buf], out_hbm.at[pl.ds(0, CHUNK)], osems[buf]
        ).wait()


def _pad_body(x_ref, o_ref):
    o_ref[:, pl.ds(0, C)] = x_ref[...]
    o_ref[:, pl.ds(C, TW - C)] = jnp.zeros((2048, TW - C), jnp.float32)


def _pad_channels(x):
    return pl.pallas_call(
        _pad_body,
        grid=(B * H * W // 2048,),
        in_specs=[pl.BlockSpec((2048, C), lambda i: (i, 0))],
        out_specs=pl.BlockSpec((2048, TW), lambda i: (i, 0)),
        out_shape=jax.ShapeDtypeStruct((B * H * W, TW), jnp.float32),
    )(x)


def kernel(images, theta):
    img = _pad_channels(images.reshape(B * H * W, C))
    theta_r = theta.astype(jnp.bfloat16).astype(jnp.float32)
    theta_p = jnp.pad(theta_r, ((0, 0), (0, 10)))
    mesh = plsc.VectorSubcoreMesh(core_axis_name="c", subcore_axis_name="s")
    k = pl.kernel(
        _body,
        out_type=jax.ShapeDtypeStruct((B * H * W, C), jnp.float32),
        mesh=mesh,
        compiler_params=pltpu.CompilerParams(use_tc_tiling_on_sc=False),
        scratch_types=[
            pltpu.VMEM((B, 16), jnp.float32),
            pltpu.VMEM((2, 4, CHUNK), jnp.int32),
            pltpu.VMEM((2, 4, CHUNK + 16), jnp.float32),
            pltpu.VMEM((2, 4 * CHUNK, TW), jnp.float32),
            pltpu.VMEM((2, CHUNK, C), jnp.float32),
            pltpu.SemaphoreType.DMA,
            pltpu.SemaphoreType.DMA,
            pltpu.SemaphoreType.DMA,
            pltpu.SemaphoreType.DMA,
        ],
    )
    out = k(img, theta_p)
    return out.reshape(B, H, W, C)


# speedup vs baseline: 1.2274x; 1.2274x over previous
import functools

import jax
import jax.numpy as jnp
from jax import lax
from jax.experimental import pallas as pl
from jax.experimental.pallas import tpu as pltpu
from jax.experimental.pallas import tpu_sc as plsc

B = 8
H = 224
W = 224
C = 96
NC = 2
NS = 16
NW = NC * NS
ROWS_PER_W = H * B // NW
CHUNK = 112
PARTS = W // CHUNK
OW = 128
CG = C // 16
NCHUNK = PARTS * ROWS_PER_W


def _bcast_f32(x):
    return lax.broadcast_in_dim(x, (16,), ())


def _body(img_hbm, theta_hbm, out_hbm, theta_v, idx_v, w_v, rows_v, outb_v,
          gsem0, gsem1, osem0, osem1):
    wid = lax.axis_index("s") * NC + lax.axis_index("c")
    b = wid // 4
    sub = wid % 4

    pltpu.sync_copy(theta_hbm, theta_v)
    tv = theta_v[b, :]
    t0 = _bcast_f32(tv[0])
    t1 = _bcast_f32(tv[1])
    t2 = _bcast_f32(tv[2])
    t3 = _bcast_f32(tv[3])
    t4 = _bcast_f32(tv[4])
    t5 = _bcast_f32(tv[5])

    lane = lax.iota(jnp.int32, 16)
    bbase = b * (H * W)
    bbase_v = lax.broadcast_in_dim(bbase, (16,), ())
    maxc = 223.0
    gsems = (gsem0, gsem1)
    osems = (osem0, osem1)

    def produce(cc, buf):
        i = (cc // PARTS) * 4 + sub
        part = cc % PARTS
        iv = lax.broadcast_in_dim(i, (16,), ())
        ifv = iv.astype(jnp.float32)
        jbase = part * CHUNK
        for g in range(CHUNK // 16):
            jv = lane + jbase + g * 16
            jf = jv.astype(jnp.float32)
            ys = t0 * ifv + t1 * jf + t2
            xs = t3 * ifv + t4 * jf + t5
            ys0 = jnp.maximum(ys, 0.0)
            xs0 = jnp.maximum(xs, 0.0)
            yl = ys0.astype(jnp.int32)
            xl = xs0.astype(jnp.int32)
            dy = ys0 - yl.astype(jnp.float32)
            dx = xs0 - xl.astype(jnp.float32)
            ylc = jnp.minimum(yl, 223)
            yuc = jnp.minimum(yl + 1, 223)
            xlc = jnp.minimum(xl, 223)
            xuc = jnp.minimum(xl + 1, 223)
            ok = ((ys >= 0.0) & (ys <= maxc)) & ((xs >= 0.0) & (xs <= maxc))
            m = jnp.where(ok, 1.0, 0.0).astype(jnp.float32)
            ny = 1.0 - dy
            nx = 1.0 - dx
            r0 = bbase_v + ylc * W
            r1 = bbase_v + yuc * W
            s = pl.ds(g * 16, 16)
            idx_v[buf, 0, s] = r0 + xlc
            idx_v[buf, 1, s] = r0 + xuc
            idx_v[buf, 2, s] = r1 + xlc
            idx_v[buf, 3, s] = r1 + xuc
            w_v[buf, 0, s] = nx * ny * m
            w_v[buf, 1, s] = dx * ny * m
            w_v[buf, 2, s] = nx * dy * m
            w_v[buf, 3, s] = dx * dy * m
        for c in range(4):
            pltpu.async_copy(
                img_hbm.at[idx_v.at[buf, c]],
                rows_v.at[buf, pl.ds(c * CHUNK, CHUNK)],
                gsems[buf],
            )

    def consume(cc, buf):
        @pl.when(cc >= 2)
        def _():
            pltpu.make_async_copy(
                outb_v.at[buf], out_hbm.at[pl.ds(0, CHUNK)], osems[buf]
            ).wait()

        for c in range(4):
            pltpu.make_async_copy(
                img_hbm.at[idx_v.at[buf, c]],
                rows_v.at[buf, pl.ds(c * CHUNK, CHUNK)],
                gsems[buf],
            ).wait()

        @pl.loop(0, CHUNK // 16)
        def _grp(g):
            base = g * 16
            ws = pl.ds(base, 16)
            w0v = w_v[buf, 0, ws]
            w1v = w_v[buf, 1, ws]
            w2v = w_v[buf, 2, ws]
            w3v = w_v[buf, 3, ws]
            for p16 in range(16):
                p = base + p16
                w0 = w0v[p16]
                w1 = w1v[p16]
                w2 = w2v[p16]
                w3 = w3v[p16]
                for cg in range(CG):
                    cs = pl.ds(cg * 16, 16)
                    v = (rows_v[buf, p, cs] * w0
                         + rows_v[buf, CHUNK + p, cs] * w1
                         + rows_v[buf, 2 * CHUNK + p, cs] * w2
                         + rows_v[buf, 3 * CHUNK + p, cs] * w3)
                    outb_v[buf, p, cs] = v

        i = (cc // PARTS) * 4 + sub
        part = cc % PARTS
        gbase = bbase + i * W + part * CHUNK
        pltpu.async_copy(
            outb_v.at[buf], out_hbm.at[pl.ds(gbase, CHUNK)], osems[buf]
        )

    produce(0, 0)

    @pl.loop(0, NCHUNK // 2 - 1)
    def _t(t):
        cc = 2 * t
        produce(cc + 1, 1)
        consume(cc, 0)
        produce(cc + 2, 0)
        consume(cc + 1, 1)

    produce(NCHUNK - 1, 1)
    consume(NCHUNK - 2, 0)
    consume(NCHUNK - 1, 1)

    for buf in range(2):
        pltpu.make_async_copy(
            outb_v.at[buf], out_hbm.at[pl.ds(0, CHUNK)], osems[buf]
        ).wait()


def kernel(images, theta):
    img = images.reshape(B * H * W, C)
    theta_r = theta.astype(jnp.bfloat16).astype(jnp.float32)
    theta_p = jnp.pad(theta_r, ((0, 0), (0, 10)))
    mesh = plsc.VectorSubcoreMesh(core_axis_name="c", subcore_axis_name="s")
    k = pl.kernel(
        _body,
        out_type=jax.ShapeDtypeStruct((B * H * W, OW), jnp.float32),
        mesh=mesh,
        compiler_params=pltpu.CompilerParams(use_tc_tiling_on_sc=False),
        scratch_types=[
            pltpu.VMEM((B, 16), jnp.float32),
            pltpu.VMEM((2, 4, CHUNK), jnp.int32),
            pltpu.VMEM((2, 4, CHUNK + 16), jnp.float32),
            pltpu.VMEM((2, 4 * CHUNK, C), jnp.float32),
            pltpu.VMEM((2, CHUNK, OW), jnp.float32),
            pltpu.SemaphoreType.DMA,
            pltpu.SemaphoreType.DMA,
            pltpu.SemaphoreType.DMA,
            pltpu.SemaphoreType.DMA,
        ],
    )
    out = k(img, theta_p)
    return out[:, :C].reshape(B, H, W, C)


# speedup vs baseline: 1.2316x; 1.0034x over previous
import jax
import jax.numpy as jnp
from jax import lax
from jax.experimental import pallas as pl
from jax.experimental.pallas import tpu as pltpu
from jax.experimental.pallas import tpu_sc as plsc

B = 8
H = 224
W = 224
C = 96
NC = 2
NS = 16
NW = NC * NS
ROWS_PER_W = H * B // NW
CHUNK = 112
PARTS = W // CHUNK
OW = 128
CG = C // 16
NCHUNK = PARTS * ROWS_PER_W


def _bcast_f32(x):
    return lax.broadcast_in_dim(x, (16,), ())


def _body(img_hbm, theta_hbm, out_hbm, theta_v, idx_v, w_v, rows_v, outb_v,
          gsem0, gsem1, osem0, osem1):
    wid = lax.axis_index("s") * NC + lax.axis_index("c")
    b = wid // 4
    sub = wid % 4

    pltpu.sync_copy(theta_hbm, theta_v)
    tv = theta_v[b, :]
    t0 = _bcast_f32(tv[0])
    t1 = _bcast_f32(tv[1])
    t2 = _bcast_f32(tv[2])
    t3 = _bcast_f32(tv[3])
    t4 = _bcast_f32(tv[4])
    t5 = _bcast_f32(tv[5])

    lane = lax.iota(jnp.int32, 16)
    bbase = b * (H * W)
    bbase_v = lax.broadcast_in_dim(bbase, (16,), ())
    maxc = 223.0
    gsems = (gsem0, gsem1)
    osems = (osem0, osem1)

    def produce(cc, buf):
        i = (cc // PARTS) * 4 + sub
        part = cc % PARTS
        iv = lax.broadcast_in_dim(i, (16,), ())
        ifv = iv.astype(jnp.float32)
        jbase = part * CHUNK
        for g in range(CHUNK // 16):
            jv = lane + jbase + g * 16
            jf = jv.astype(jnp.float32)
            ys = t0 * ifv + t1 * jf + t2
            xs = t3 * ifv + t4 * jf + t5
            ys0 = jnp.maximum(ys, 0.0)
            xs0 = jnp.maximum(xs, 0.0)
            yl = ys0.astype(jnp.int32)
            xl = xs0.astype(jnp.int32)
            dy = ys0 - yl.astype(jnp.float32)
            dx = xs0 - xl.astype(jnp.float32)
            ylc = jnp.minimum(yl, 223)
            yuc = jnp.minimum(yl + 1, 223)
            xlc = jnp.minimum(xl, 223)
            xuc = jnp.minimum(xl + 1, 223)
            ok = ((ys >= 0.0) & (ys <= maxc)) & ((xs >= 0.0) & (xs <= maxc))
            m = jnp.where(ok, 1.0, 0.0).astype(jnp.float32)
            ny = 1.0 - dy
            nx = 1.0 - dx
            r0 = bbase_v + ylc * W
            r1 = bbase_v + yuc * W
            s = pl.ds(g * 16, 16)
            idx_v[buf, 0, s] = r0 + xlc
            idx_v[buf, 1, s] = r0 + xuc
            idx_v[buf, 2, s] = r1 + xlc
            idx_v[buf, 3, s] = r1 + xuc
            w_v[buf, 0, s] = nx * ny * m
            w_v[buf, 1, s] = dx * ny * m
            w_v[buf, 2, s] = nx * dy * m
            w_v[buf, 3, s] = dx * dy * m
        for c in range(4):
            pltpu.async_copy(
                img_hbm.at[idx_v.at[buf, c]],
                rows_v.at[buf, pl.ds(c * CHUNK, CHUNK)],
                gsems[buf],
            )

    def consume(cc, buf):
        @pl.when(cc >= 2)
        def _():
            pltpu.make_async_copy(
                outb_v.at[buf], out_hbm.at[pl.ds(0, CHUNK)], osems[buf]
            ).wait()

        for c in range(4):
            pltpu.make_async_copy(
                img_hbm.at[idx_v.at[buf, c]],
                rows_v.at[buf, pl.ds(c * CHUNK, CHUNK)],
                gsems[buf],
            ).wait()

        @pl.loop(0, CHUNK // 16)
        def _grp(g):
            base = g * 16
            ws = pl.ds(base, 16)
            w0v = w_v[buf, 0, ws]
            w1v = w_v[buf, 1, ws]
            w2v = w_v[buf, 2, ws]
            w3v = w_v[buf, 3, ws]
            for p16 in range(16):
                p = base + p16
                w0 = w0v[p16]
                w1 = w1v[p16]
                w2 = w2v[p16]
                w3 = w3v[p16]
                for cg in range(CG):
                    cs = pl.ds(cg * 16, 16)
                    v = (rows_v[buf, p, cs] * w0
                         + rows_v[buf, CHUNK + p, cs] * w1
                         + rows_v[buf, 2 * CHUNK + p, cs] * w2
                         + rows_v[buf, 3 * CHUNK + p, cs] * w3)
                    outb_v[buf, p, cs] = v

        i = (cc // PARTS) * 4 + sub
        part = cc % PARTS
        gbase = bbase + i * W + part * CHUNK
        pltpu.async_copy(
            outb_v.at[buf], out_hbm.at[pl.ds(gbase, CHUNK)], osems[buf]
        )

    produce(0, 0)

    @pl.loop(0, NCHUNK // 2 - 1)
    def _t(t):
        cc = 2 * t
        produce(cc + 1, 1)
        consume(cc, 0)
        produce(cc + 2, 0)
        consume(cc + 1, 1)

    produce(NCHUNK - 1, 1)
    consume(NCHUNK - 2, 0)
    consume(NCHUNK - 1, 1)

    for buf in range(2):
        pltpu.make_async_copy(
            outb_v.at[buf], out_hbm.at[pl.ds(0, CHUNK)], osems[buf]
        ).wait()


def kernel(images, theta):
    img = images.reshape(B * H * W, C)
    theta_r = theta.astype(jnp.bfloat16).astype(jnp.float32)
    theta_p = jnp.pad(theta_r, ((0, 0), (0, 10)))
    mesh = plsc.VectorSubcoreMesh(core_axis_name="c", subcore_axis_name="s")
    k = pl.kernel(
        _body,
        out_type=jax.ShapeDtypeStruct((B * H * W, OW), jnp.float32),
        mesh=mesh,
        compiler_params=pltpu.CompilerParams(use_tc_tiling_on_sc=False),
        scratch_types=[
            pltpu.VMEM((B, 16), jnp.float32),
            pltpu.VMEM((2, 4, CHUNK), jnp.int32),
            pltpu.VMEM((2, 4, CHUNK + 16), jnp.float32),
            pltpu.VMEM((2, 4 * CHUNK, C), jnp.float32),
            pltpu.VMEM((2, CHUNK, OW), jnp.float32),
            pltpu.SemaphoreType.DMA,
            pltpu.SemaphoreType.DMA,
            pltpu.SemaphoreType.DMA,
            pltpu.SemaphoreType.DMA,
        ],
    )
    out = k(img, theta_p)
    return out[:, :C].reshape(B, H, W, C)
